# baseline (speedup 1.0000x reference)
import functools

import jax
import jax.numpy as jnp
from jax import lax
from jax.experimental import pallas as pl
from jax.experimental.pallas import tpu as pltpu
from jax.experimental.pallas import tpu_sc as plsc

BATCH = 16384
EMB_K = 64
NROWS = 100000
PANEL = 256
N_PANELS = (NROWS + PANEL - 1) // PANEL
TROWS = N_PANELS * PANEL // 2
_INFO = plsc.get_sparse_core_info()
NC, NS, L = _INFO.num_cores, _INFO.num_subcores, _INFO.num_lanes
NW = NC * NS
B_PER_W = BATCH // NW
CHUNK = 128
N_CHUNKS = B_PER_W // CHUNK
G_PER_CHUNK = CHUNK // L
K_CH = EMB_K // L


def _to_row_major(Wt):
    def body(x_ref, o_ref):
        o_ref[:, 0:EMB_K] = x_ref[:, 0:CHUNK].T
        o_ref[:, EMB_K:2 * EMB_K] = x_ref[:, CHUNK:PANEL].T

    out = pl.pallas_call(
        body,
        grid=(N_PANELS,),
        in_specs=[pl.BlockSpec((EMB_K, PANEL), lambda i: (0, i))],
        out_specs=pl.BlockSpec((CHUNK, 2 * EMB_K), lambda i: (i, 0)),
        out_shape=jax.ShapeDtypeStruct((TROWS, 2 * EMB_K), jnp.float32),
    )(Wt)
    return out.reshape(-1)


def _flat_offsets(idx):
    return (((idx >> 8) << 15) + ((idx & 127) << 7)
            + (((idx >> 7) & 1) << 6))


def _mf_dot(u_idx, v_idx, W_flat, H_flat):
    mesh = plsc.VectorSubcoreMesh(core_axis_name="c", subcore_axis_name="s")

    @functools.partial(
        pl.kernel,
        mesh=mesh,
        out_type=jax.ShapeDtypeStruct((BATCH,), jnp.float32),
        compiler_params=pltpu.CompilerParams(use_tc_tiling_on_sc=False),
        scratch_types=[
            pltpu.VMEM((B_PER_W,), jnp.int32),
            pltpu.VMEM((B_PER_W,), jnp.int32),
            pltpu.VMEM((CHUNK * EMB_K,), jnp.float32),
            pltpu.VMEM((CHUNK * EMB_K,), jnp.float32),
            pltpu.VMEM((CHUNK * EMB_K,), jnp.float32),
            pltpu.VMEM((CHUNK * EMB_K,), jnp.float32),
            pltpu.VMEM((B_PER_W,), jnp.float32),
            pltpu.VMEM((2 * L,), jnp.float32),
            pltpu.SemaphoreType.DMA,
            pltpu.SemaphoreType.DMA,
        ],
    )
    def k(u_idx_hbm, v_idx_hbm, w_hbm, h_hbm, out_hbm,
          idx_u_v, idx_v_v, u_buf0, u_buf1, v_buf0, v_buf1,
          out_v, tree_buf, sem_u, sem_v):
        c = lax.axis_index("c")
        s = lax.axis_index("s")
        wid = s * NC + c
        base_b = pl.multiple_of(wid * B_PER_W, B_PER_W)

        pltpu.sync_copy(u_idx_hbm.at[pl.ds(base_b, B_PER_W)], idx_u_v)
        pltpu.sync_copy(v_idx_hbm.at[pl.ds(base_b, B_PER_W)], idx_v_v)

        iota16 = lax.iota(jnp.int32, L)
        tree_buf[pl.ds(L, L)] = jnp.zeros((L,), jnp.float32)

        u_bufs = (u_buf0, u_buf1)
        v_bufs = (v_buf0, v_buf1)

        def fire(chunk):
            ub = u_bufs[chunk % 2]
            vb = v_bufs[chunk % 2]
            base = chunk * CHUNK

            def body(g16, carry):
                off = pl.multiple_of(g16 * L, L)
                uo = _flat_offsets(idx_u_v[pl.ds(base + off, L)])
                vo = _flat_offsets(idx_v_v[pl.ds(base + off, L)])
                for lane in range(L):
                    dst = pl.ds(
                        pl.multiple_of((off + lane) * EMB_K, EMB_K), EMB_K)
                    pltpu.async_copy(
                        w_hbm.at[pl.ds(pl.multiple_of(uo[lane], EMB_K),
                                       EMB_K)], ub.at[dst], sem_u)
                    pltpu.async_copy(
                        h_hbm.at[pl.ds(pl.multiple_of(vo[lane], EMB_K),
                                       EMB_K)], vb.at[dst], sem_v)
                return carry

            lax.fori_loop(0, CHUNK // L, body, 0)

        def drain(chunk):
            pltpu.make_async_copy(
                w_hbm.at[pl.ds(0, CHUNK * EMB_K)],
                u_bufs[chunk % 2], sem_u).wait()
            pltpu.make_async_copy(
                h_hbm.at[pl.ds(0, CHUNK * EMB_K)],
                v_bufs[chunk % 2], sem_v).wait()

        def compute(chunk):
            ub = u_bufs[chunk % 2]
            vb = v_bufs[chunk % 2]
            base = chunk * CHUNK

            def group(g, carry):
                acc = jnp.zeros((L,), jnp.float32)
                for r in range(L):
                    row_off = pl.multiple_of((g * L + r) * EMB_K, EMB_K)
                    p = (ub[pl.ds(row_off, L)] *
                         vb[pl.ds(row_off, L)])
                    for cch in range(1, K_CH):
                        sl = pl.ds(
                            pl.multiple_of(row_off + cch * L, L), L)
                        p = p + ub[sl] * vb[sl]
                    tree_buf[pl.ds(0, L)] = p
                    t = p
                    for sh in (8, 4, 2, 1):
                        t = t + tree_buf[pl.ds(sh, L)]
                        tree_buf[pl.ds(0, L)] = t
                    acc = jnp.where(
                        iota16 == r, lax.broadcast(t[0], (L,)), acc)
                out_v[pl.ds(pl.multiple_of(base + g * L, L), L)] = acc
                return carry

            lax.fori_loop(0, G_PER_CHUNK, group, 0)

        fire(0)
        fire(1)
        for chunk in range(N_CHUNKS):
            drain(chunk)
            compute(chunk)
            if chunk + 2 < N_CHUNKS:
                fire(chunk + 2)

        pltpu.sync_copy(out_v, out_hbm.at[pl.ds(base_b, B_PER_W)])

    return k(u_idx, v_idx, W_flat, H_flat)


def kernel(x, W, H, W_pre, H_pre, W_eps, H_eps):
    xi = x.astype(jnp.int32)
    u_idx = xi[:, 0]
    v_idx = xi[:, 1]
    W_flat = _to_row_major(W.T)
    H_flat = _to_row_major(H.T)
    return _mf_dot(u_idx, v_idx, W_flat, H_flat)


# speedup vs baseline: 1.0008x; 1.0008x over previous
import functools

import jax
import jax.numpy as jnp
from jax import lax
from jax.experimental import pallas as pl
from jax.experimental.pallas import tpu as pltpu
from jax.experimental.pallas import tpu_sc as plsc

BATCH = 16384
EMB_K = 64
NROWS = 100000
PANEL = 256
N_PANELS = (NROWS + PANEL - 1) // PANEL
TROWS = N_PANELS * PANEL // 2
_INFO = plsc.get_sparse_core_info()
NC, NS, L = _INFO.num_cores, _INFO.num_subcores, _INFO.num_lanes
NW = NC * NS
B_PER_W = BATCH // NW
CHUNK = 128
N_CHUNKS = B_PER_W // CHUNK
G_PER_CHUNK = CHUNK // L
K_CH = EMB_K // L


def _to_row_major(Wt):
    def body(x_ref, o_ref):
        o_ref[:, 0:EMB_K] = x_ref[:, 0:CHUNK].T
        o_ref[:, EMB_K:2 * EMB_K] = x_ref[:, CHUNK:PANEL].T

    out = pl.pallas_call(
        body,
        grid=(N_PANELS,),
        in_specs=[pl.BlockSpec((EMB_K, PANEL), lambda i: (0, i))],
        out_specs=pl.BlockSpec((CHUNK, 2 * EMB_K), lambda i: (i, 0)),
        out_shape=jax.ShapeDtypeStruct((TROWS, 2 * EMB_K), jnp.float32),
    )(Wt)
    return out.reshape(-1)


def _flat_offsets(idx):
    return (((idx >> 8) << 14) + ((idx & 127) << 7)
            + (((idx >> 7) & 1) << 6))


def _mf_dot(u_idx, v_idx, W_flat, H_flat):
    mesh = plsc.VectorSubcoreMesh(core_axis_name="c", subcore_axis_name="s")

    @functools.partial(
        pl.kernel,
        mesh=mesh,
        out_type=jax.ShapeDtypeStruct((BATCH,), jnp.float32),
        compiler_params=pltpu.CompilerParams(use_tc_tiling_on_sc=False),
        scratch_types=[
            pltpu.VMEM((B_PER_W,), jnp.int32),
            pltpu.VMEM((B_PER_W,), jnp.int32),
            pltpu.VMEM((CHUNK * EMB_K,), jnp.float32),
            pltpu.VMEM((CHUNK * EMB_K,), jnp.float32),
            pltpu.VMEM((CHUNK * EMB_K,), jnp.float32),
            pltpu.VMEM((CHUNK * EMB_K,), jnp.float32),
            pltpu.VMEM((B_PER_W,), jnp.float32),
            pltpu.VMEM((2 * L,), jnp.float32),
            pltpu.SemaphoreType.DMA,
            pltpu.SemaphoreType.DMA,
        ],
    )
    def k(u_idx_hbm, v_idx_hbm, w_hbm, h_hbm, out_hbm,
          idx_u_v, idx_v_v, u_buf0, u_buf1, v_buf0, v_buf1,
          out_v, tree_buf, sem_u, sem_v):
        c = lax.axis_index("c")
        s = lax.axis_index("s")
        wid = s * NC + c
        base_b = pl.multiple_of(wid * B_PER_W, B_PER_W)

        pltpu.sync_copy(u_idx_hbm.at[pl.ds(base_b, B_PER_W)], idx_u_v)
        pltpu.sync_copy(v_idx_hbm.at[pl.ds(base_b, B_PER_W)], idx_v_v)

        iota16 = lax.iota(jnp.int32, L)
        tree_buf[pl.ds(L, L)] = jnp.zeros((L,), jnp.float32)

        u_bufs = (u_buf0, u_buf1)
        v_bufs = (v_buf0, v_buf1)

        def fire(chunk):
            ub = u_bufs[chunk % 2]
            vb = v_bufs[chunk % 2]
            base = chunk * CHUNK

            def body(g16, carry):
                off = pl.multiple_of(g16 * L, L)
                uo = _flat_offsets(idx_u_v[pl.ds(base + off, L)])
                vo = _flat_offsets(idx_v_v[pl.ds(base + off, L)])
                for lane in range(L):
                    dst = pl.ds(
                        pl.multiple_of((off + lane) * EMB_K, EMB_K), EMB_K)
                    pltpu.async_copy(
                        w_hbm.at[pl.ds(pl.multiple_of(uo[lane], EMB_K),
                                       EMB_K)], ub.at[dst], sem_u)
                    pltpu.async_copy(
                        h_hbm.at[pl.ds(pl.multiple_of(vo[lane], EMB_K),
                                       EMB_K)], vb.at[dst], sem_v)
                return carry

            lax.fori_loop(0, CHUNK // L, body, 0)

        def drain(chunk):
            pltpu.make_async_copy(
                w_hbm.at[pl.ds(0, CHUNK * EMB_K)],
                u_bufs[chunk % 2], sem_u).wait()
            pltpu.make_async_copy(
                h_hbm.at[pl.ds(0, CHUNK * EMB_K)],
                v_bufs[chunk % 2], sem_v).wait()

        def compute(chunk):
            ub = u_bufs[chunk % 2]
            vb = v_bufs[chunk % 2]
            base = chunk * CHUNK

            def group(g, carry):
                acc = jnp.zeros((L,), jnp.float32)
                for r in range(L):
                    row_off = pl.multiple_of((g * L + r) * EMB_K, EMB_K)
                    p = (ub[pl.ds(row_off, L)] *
                         vb[pl.ds(row_off, L)])
                    for cch in range(1, K_CH):
                        sl = pl.ds(
                            pl.multiple_of(row_off + cch * L, L), L)
                        p = p + ub[sl] * vb[sl]
                    tree_buf[pl.ds(0, L)] = p
                    t = p
                    for sh in (8, 4, 2, 1):
                        t = t + tree_buf[pl.ds(sh, L)]
                        tree_buf[pl.ds(0, L)] = t
                    acc = jnp.where(
                        iota16 == r, lax.broadcast(t[0], (L,)), acc)
                out_v[pl.ds(pl.multiple_of(base + g * L, L), L)] = acc
                return carry

            lax.fori_loop(0, G_PER_CHUNK, group, 0)

        fire(0)
        fire(1)
        for chunk in range(N_CHUNKS):
            drain(chunk)
            compute(chunk)
            if chunk + 2 < N_CHUNKS:
                fire(chunk + 2)

        pltpu.sync_copy(out_v, out_hbm.at[pl.ds(base_b, B_PER_W)])

    return k(u_idx, v_idx, W_flat, H_flat)


def kernel(x, W, H, W_pre, H_pre, W_eps, H_eps):
    xi = x.astype(jnp.int32)
    u_idx = xi[:, 0]
    v_idx = xi[:, 1]
    W_flat = _to_row_major(W.T)
    H_flat = _to_row_major(H.T)
    return _mf_dot(u_idx, v_idx, W_flat, H_flat)


# speedup vs baseline: 3.6940x; 3.6910x over previous
import functools

import jax
import jax.numpy as jnp
from jax import lax
from jax.experimental import pallas as pl
from jax.experimental.pallas import tpu as pltpu
from jax.experimental.pallas import tpu_sc as plsc

BATCH = 16384
EMB_K = 64
NROWS = 100000
PANEL = 256
TBLK = 2048
N_TBLK = (NROWS + TBLK - 1) // TBLK
TROWS = N_TBLK * TBLK // 2
_INFO = plsc.get_sparse_core_info()
NC, NS, L = _INFO.num_cores, _INFO.num_subcores, _INFO.num_lanes
NW = NC * NS
B_PER_W = BATCH // NW
CHUNK = 128
N_CHUNKS = B_PER_W // CHUNK
G_PER_CHUNK = CHUNK // L
K_CH = EMB_K // L


def _to_row_major(Wt):
    def body(x_ref, o_ref):
        ident = (lax.broadcasted_iota(jnp.int32, (EMB_K, EMB_K), 0)
                 == lax.broadcasted_iota(jnp.int32, (EMB_K, EMB_K), 1)
                 ).astype(jnp.float32)
        dn = (((0,), (0,)), ((), ()))
        for p in range(TBLK // PANEL):
            for half in range(2):
                src = pl.ds(p * PANEL + half * CHUNK, CHUNK)
                o_ref[pl.ds(p * CHUNK, CHUNK),
                      pl.ds(half * EMB_K, EMB_K)] = lax.dot_general(
                    x_ref[:, src], ident, dn,
                    preferred_element_type=jnp.float32)

    out = pl.pallas_call(
        body,
        grid=(N_TBLK,),
        in_specs=[pl.BlockSpec((EMB_K, TBLK), lambda i: (0, i))],
        out_specs=pl.BlockSpec((TBLK // 2, 2 * EMB_K), lambda i: (i, 0)),
        out_shape=jax.ShapeDtypeStruct((TROWS, 2 * EMB_K), jnp.float32),
    )(Wt)
    return out.reshape(-1)


def _flat_offsets(idx):
    return (((idx >> 8) << 14) + ((idx & 127) << 7)
            + (((idx >> 7) & 1) << 6))


def _mf_dot(u_idx, v_idx, W_flat, H_flat):
    mesh = plsc.VectorSubcoreMesh(core_axis_name="c", subcore_axis_name="s")

    @functools.partial(
        pl.kernel,
        mesh=mesh,
        out_type=jax.ShapeDtypeStruct((BATCH,), jnp.float32),
        compiler_params=pltpu.CompilerParams(use_tc_tiling_on_sc=False),
        scratch_types=[
            pltpu.VMEM((B_PER_W,), jnp.int32),
            pltpu.VMEM((B_PER_W,), jnp.int32),
            pltpu.VMEM((CHUNK * EMB_K,), jnp.float32),
            pltpu.VMEM((CHUNK * EMB_K,), jnp.float32),
            pltpu.VMEM((CHUNK * EMB_K,), jnp.float32),
            pltpu.VMEM((CHUNK * EMB_K,), jnp.float32),
            pltpu.VMEM((B_PER_W,), jnp.float32),
            pltpu.VMEM((2 * L,), jnp.float32),
            pltpu.SemaphoreType.DMA,
            pltpu.SemaphoreType.DMA,
        ],
    )
    def k(u_idx_hbm, v_idx_hbm, w_hbm, h_hbm, out_hbm,
          idx_u_v, idx_v_v, u_buf0, u_buf1, v_buf0, v_buf1,
          out_v, tree_buf, sem_u, sem_v):
        c = lax.axis_index("c")
        s = lax.axis_index("s")
        wid = s * NC + c
        base_b = pl.multiple_of(wid * B_PER_W, B_PER_W)

        pltpu.sync_copy(u_idx_hbm.at[pl.ds(base_b, B_PER_W)], idx_u_v)
        pltpu.sync_copy(v_idx_hbm.at[pl.ds(base_b, B_PER_W)], idx_v_v)

        iota16 = lax.iota(jnp.int32, L)
        tree_buf[pl.ds(L, L)] = jnp.zeros((L,), jnp.float32)

        u_bufs = (u_buf0, u_buf1)
        v_bufs = (v_buf0, v_buf1)

        def fire(chunk):
            ub = u_bufs[chunk % 2]
            vb = v_bufs[chunk % 2]
            base = chunk * CHUNK

            def body(g16, carry):
                off = pl.multiple_of(g16 * L, L)
                uo = _flat_offsets(idx_u_v[pl.ds(base + off, L)])
                vo = _flat_offsets(idx_v_v[pl.ds(base + off, L)])
                for lane in range(L):
                    dst = pl.ds(
                        pl.multiple_of((off + lane) * EMB_K, EMB_K), EMB_K)
                    pltpu.async_copy(
                        w_hbm.at[pl.ds(pl.multiple_of(uo[lane], EMB_K),
                                       EMB_K)], ub.at[dst], sem_u)
                    pltpu.async_copy(
                        h_hbm.at[pl.ds(pl.multiple_of(vo[lane], EMB_K),
                                       EMB_K)], vb.at[dst], sem_v)
                return carry

            lax.fori_loop(0, CHUNK // L, body, 0)

        def drain(chunk):
            pltpu.make_async_copy(
                w_hbm.at[pl.ds(0, CHUNK * EMB_K)],
                u_bufs[chunk % 2], sem_u).wait()
            pltpu.make_async_copy(
                h_hbm.at[pl.ds(0, CHUNK * EMB_K)],
                v_bufs[chunk % 2], sem_v).wait()

        def compute(chunk):
            ub = u_bufs[chunk % 2]
            vb = v_bufs[chunk % 2]
            base = chunk * CHUNK

            def group(g, carry):
                acc = jnp.zeros((L,), jnp.float32)
                for r in range(L):
                    row_off = pl.multiple_of((g * L + r) * EMB_K, EMB_K)
                    p = (ub[pl.ds(row_off, L)] *
                         vb[pl.ds(row_off, L)])
                    for cch in range(1, K_CH):
                        sl = pl.ds(
                            pl.multiple_of(row_off + cch * L, L), L)
                        p = p + ub[sl] * vb[sl]
                    tree_buf[pl.ds(0, L)] = p
                    t = p
                    for sh in (8, 4, 2, 1):
                        t = t + tree_buf[pl.ds(sh, L)]
                        tree_buf[pl.ds(0, L)] = t
                    acc = jnp.where(
                        iota16 == r, lax.broadcast(t[0], (L,)), acc)
                out_v[pl.ds(pl.multiple_of(base + g * L, L), L)] = acc
                return carry

            lax.fori_loop(0, G_PER_CHUNK, group, 0)

        fire(0)
        fire(1)
        for chunk in range(N_CHUNKS):
            drain(chunk)
            compute(chunk)
            if chunk + 2 < N_CHUNKS:
                fire(chunk + 2)

        pltpu.sync_copy(out_v, out_hbm.at[pl.ds(base_b, B_PER_W)])

    return k(u_idx, v_idx, W_flat, H_flat)


def kernel(x, W, H, W_pre, H_pre, W_eps, H_eps):
    xi = x.astype(jnp.int32)
    u_idx = xi[:, 0]
    v_idx = xi[:, 1]
    W_flat = _to_row_major(W.T)
    H_flat = _to_row_major(H.T)
    return _mf_dot(u_idx, v_idx, W_flat, H_flat)


# speedup vs baseline: 3.9760x; 1.0763x over previous
import functools

import jax
import jax.numpy as jnp
from jax import lax
from jax.experimental import pallas as pl
from jax.experimental.pallas import tpu as pltpu
from jax.experimental.pallas import tpu_sc as plsc

BATCH = 16384
EMB_K = 64
NROWS = 100000
PANEL = 256
TBLK = 2048
N_TBLK = (NROWS + TBLK - 1) // TBLK
TROWS = N_TBLK * TBLK // 2
_INFO = plsc.get_sparse_core_info()
NC, NS, L = _INFO.num_cores, _INFO.num_subcores, _INFO.num_lanes
NW = NC * NS
B_PER_W = BATCH // NW
CHUNK = 128
N_CHUNKS = B_PER_W // CHUNK
G_PER_CHUNK = CHUNK // L
K_CH = EMB_K // L


def _to_row_major(Wt):
    def body(x_ref, o_ref):
        ident = (lax.broadcasted_iota(jnp.int32, (CHUNK, CHUNK), 0)
                 == lax.broadcasted_iota(jnp.int32, (CHUNK, CHUNK), 1)
                 ).astype(jnp.float32)
        dn = (((0,), (0,)), ((), ()))
        for p in range(TBLK // PANEL):
            lhs = jnp.concatenate(
                [x_ref[:, pl.ds(p * PANEL, CHUNK)],
                 x_ref[:, pl.ds(p * PANEL + CHUNK, CHUNK)]], axis=0)
            o_ref[pl.ds(p * CHUNK, CHUNK), :] = lax.dot_general(
                lhs, ident, dn, preferred_element_type=jnp.float32)

    out = pl.pallas_call(
        body,
        grid=(N_TBLK,),
        in_specs=[pl.BlockSpec((EMB_K, TBLK), lambda i: (0, i))],
        out_specs=pl.BlockSpec((TBLK // 2, 2 * EMB_K), lambda i: (i, 0)),
        out_shape=jax.ShapeDtypeStruct((TROWS, 2 * EMB_K), jnp.float32),
    )(Wt)
    return out.reshape(-1)


def _flat_offsets(idx):
    return (((idx >> 8) << 14) + ((idx & 127) << 7)
            + (((idx >> 7) & 1) << 6))


def _mf_dot(u_idx, v_idx, W_flat, H_flat):
    mesh = plsc.VectorSubcoreMesh(core_axis_name="c", subcore_axis_name="s")

    @functools.partial(
        pl.kernel,
        mesh=mesh,
        out_type=jax.ShapeDtypeStruct((BATCH,), jnp.float32),
        compiler_params=pltpu.CompilerParams(use_tc_tiling_on_sc=False),
        scratch_types=[
            pltpu.VMEM((B_PER_W,), jnp.int32),
            pltpu.VMEM((B_PER_W,), jnp.int32),
            pltpu.VMEM((CHUNK * EMB_K,), jnp.float32),
            pltpu.VMEM((CHUNK * EMB_K,), jnp.float32),
            pltpu.VMEM((CHUNK * EMB_K,), jnp.float32),
            pltpu.VMEM((CHUNK * EMB_K,), jnp.float32),
            pltpu.VMEM((B_PER_W,), jnp.float32),
            pltpu.VMEM((2 * L,), jnp.float32),
            pltpu.SemaphoreType.DMA,
            pltpu.SemaphoreType.DMA,
        ],
    )
    def k(u_idx_hbm, v_idx_hbm, w_hbm, h_hbm, out_hbm,
          idx_u_v, idx_v_v, u_buf0, u_buf1, v_buf0, v_buf1,
          out_v, tree_buf, sem_u, sem_v):
        c = lax.axis_index("c")
        s = lax.axis_index("s")
        wid = s * NC + c
        base_b = pl.multiple_of(wid * B_PER_W, B_PER_W)

        pltpu.sync_copy(u_idx_hbm.at[pl.ds(base_b, B_PER_W)], idx_u_v)
        pltpu.sync_copy(v_idx_hbm.at[pl.ds(base_b, B_PER_W)], idx_v_v)

        iota16 = lax.iota(jnp.int32, L)
        tree_buf[pl.ds(L, L)] = jnp.zeros((L,), jnp.float32)

        u_bufs = (u_buf0, u_buf1)
        v_bufs = (v_buf0, v_buf1)

        def fire(chunk):
            ub = u_bufs[chunk % 2]
            vb = v_bufs[chunk % 2]
            base = chunk * CHUNK

            def body(g16, carry):
                off = pl.multiple_of(g16 * L, L)
                uo = _flat_offsets(idx_u_v[pl.ds(base + off, L)])
                vo = _flat_offsets(idx_v_v[pl.ds(base + off, L)])
                for lane in range(L):
                    dst = pl.ds(
                        pl.multiple_of((off + lane) * EMB_K, EMB_K), EMB_K)
                    pltpu.async_copy(
                        w_hbm.at[pl.ds(pl.multiple_of(uo[lane], EMB_K),
                                       EMB_K)], ub.at[dst], sem_u)
                    pltpu.async_copy(
                        h_hbm.at[pl.ds(pl.multiple_of(vo[lane], EMB_K),
                                       EMB_K)], vb.at[dst], sem_v)
                return carry

            lax.fori_loop(0, CHUNK // L, body, 0)

        def drain(chunk):
            pltpu.make_async_copy(
                w_hbm.at[pl.ds(0, CHUNK * EMB_K)],
                u_bufs[chunk % 2], sem_u).wait()
            pltpu.make_async_copy(
                h_hbm.at[pl.ds(0, CHUNK * EMB_K)],
                v_bufs[chunk % 2], sem_v).wait()

        def compute(chunk):
            ub = u_bufs[chunk % 2]
            vb = v_bufs[chunk % 2]
            base = chunk * CHUNK

            def group(g, carry):
                acc = jnp.zeros((L,), jnp.float32)
                for r in range(L):
                    row_off = pl.multiple_of((g * L + r) * EMB_K, EMB_K)
                    p = (ub[pl.ds(row_off, L)] *
                         vb[pl.ds(row_off, L)])
                    for cch in range(1, K_CH):
                        sl = pl.ds(
                            pl.multiple_of(row_off + cch * L, L), L)
                        p = p + ub[sl] * vb[sl]
                    tree_buf[pl.ds(0, L)] = p
                    t = p
                    for sh in (8, 4, 2, 1):
                        t = t + tree_buf[pl.ds(sh, L)]
                        tree_buf[pl.ds(0, L)] = t
                    acc = jnp.where(
                        iota16 == r, lax.broadcast(t[0], (L,)), acc)
                out_v[pl.ds(pl.multiple_of(base + g * L, L), L)] = acc
                return carry

            lax.fori_loop(0, G_PER_CHUNK, group, 0)

        fire(0)
        fire(1)
        for chunk in range(N_CHUNKS):
            drain(chunk)
            compute(chunk)
            if chunk + 2 < N_CHUNKS:
                fire(chunk + 2)

        pltpu.sync_copy(out_v, out_hbm.at[pl.ds(base_b, B_PER_W)])

    return k(u_idx, v_idx, W_flat, H_flat)


def kernel(x, W, H, W_pre, H_pre, W_eps, H_eps):
    xi = x.astype(jnp.int32)
    u_idx = xi[:, 0]
    v_idx = xi[:, 1]
    W_flat = _to_row_major(W.T)
    H_flat = _to_row_major(H.T)
    return _mf_dot(u_idx, v_idx, W_flat, H_flat)


# speedup vs baseline: 5.9149x; 1.4876x over previous
import functools

import jax
import jax.numpy as jnp
from jax import lax
from jax.experimental import pallas as pl
from jax.experimental.pallas import tpu as pltpu
from jax.experimental.pallas import tpu_sc as plsc

BATCH = 16384
EMB_K = 64
NROWS = 100000
PANEL = 256
TBLK = 8192
N_TBLK = (NROWS + TBLK - 1) // TBLK
TROWS = N_TBLK * TBLK // 2
_INFO = plsc.get_sparse_core_info()
NC, NS, L = _INFO.num_cores, _INFO.num_subcores, _INFO.num_lanes
NW = NC * NS
B_PER_W = BATCH // NW
CHUNK = 128
N_CHUNKS = B_PER_W // CHUNK
G_PER_CHUNK = CHUNK // L
K_CH = EMB_K // L


def _to_row_major(Wt):
    def body(x_ref, o_ref):
        ident = (lax.broadcasted_iota(jnp.int32, (CHUNK, CHUNK), 0)
                 == lax.broadcasted_iota(jnp.int32, (CHUNK, CHUNK), 1)
                 ).astype(jnp.float32)
        dn = (((0,), (0,)), ((), ()))
        for p in range(TBLK // PANEL):
            lhs = jnp.concatenate(
                [x_ref[:, pl.ds(p * PANEL, CHUNK)],
                 x_ref[:, pl.ds(p * PANEL + CHUNK, CHUNK)]], axis=0)
            o_ref[pl.ds(p * CHUNK, CHUNK), :] = lax.dot_general(
                lhs, ident, dn, preferred_element_type=jnp.float32)

    out = pl.pallas_call(
        body,
        grid=(N_TBLK,),
        in_specs=[pl.BlockSpec((EMB_K, TBLK), lambda i: (0, i))],
        out_specs=pl.BlockSpec((TBLK // 2, 2 * EMB_K), lambda i: (i, 0)),
        out_shape=jax.ShapeDtypeStruct((TROWS, 2 * EMB_K), jnp.float32),
    )(Wt)
    return out.reshape(-1)


def _flat_offsets(idx):
    return (((idx >> 8) << 14) + ((idx & 127) << 7)
            + (((idx >> 7) & 1) << 6))


def _mf_dot(u_idx, v_idx, W_flat, H_flat):
    mesh = plsc.VectorSubcoreMesh(core_axis_name="c", subcore_axis_name="s")

    @functools.partial(
        pl.kernel,
        mesh=mesh,
        out_type=jax.ShapeDtypeStruct((BATCH,), jnp.float32),
        compiler_params=pltpu.CompilerParams(use_tc_tiling_on_sc=False),
        scratch_types=[
            pltpu.VMEM((B_PER_W,), jnp.int32),
            pltpu.VMEM((B_PER_W,), jnp.int32),
            pltpu.VMEM((CHUNK * EMB_K,), jnp.float32),
            pltpu.VMEM((CHUNK * EMB_K,), jnp.float32),
            pltpu.VMEM((CHUNK * EMB_K,), jnp.float32),
            pltpu.VMEM((CHUNK * EMB_K,), jnp.float32),
            pltpu.VMEM((B_PER_W,), jnp.float32),
            pltpu.VMEM((2 * L,), jnp.float32),
            pltpu.SemaphoreType.DMA,
            pltpu.SemaphoreType.DMA,
        ],
    )
    def k(u_idx_hbm, v_idx_hbm, w_hbm, h_hbm, out_hbm,
          idx_u_v, idx_v_v, u_buf0, u_buf1, v_buf0, v_buf1,
          out_v, tree_buf, sem_u, sem_v):
        c = lax.axis_index("c")
        s = lax.axis_index("s")
        wid = s * NC + c
        base_b = pl.multiple_of(wid * B_PER_W, B_PER_W)

        pltpu.sync_copy(u_idx_hbm.at[pl.ds(base_b, B_PER_W)], idx_u_v)
        pltpu.sync_copy(v_idx_hbm.at[pl.ds(base_b, B_PER_W)], idx_v_v)

        iota16 = lax.iota(jnp.int32, L)
        tree_buf[pl.ds(L, L)] = jnp.zeros((L,), jnp.float32)

        u_bufs = (u_buf0, u_buf1)
        v_bufs = (v_buf0, v_buf1)

        def fire(chunk):
            ub = u_bufs[chunk % 2]
            vb = v_bufs[chunk % 2]
            base = chunk * CHUNK

            def body(g16, carry):
                off = pl.multiple_of(g16 * L, L)
                uo = _flat_offsets(idx_u_v[pl.ds(base + off, L)])
                vo = _flat_offsets(idx_v_v[pl.ds(base + off, L)])
                for lane in range(L):
                    dst = pl.ds(
                        pl.multiple_of((off + lane) * EMB_K, EMB_K), EMB_K)
                    pltpu.async_copy(
                        w_hbm.at[pl.ds(pl.multiple_of(uo[lane], EMB_K),
                                       EMB_K)], ub.at[dst], sem_u)
                    pltpu.async_copy(
                        h_hbm.at[pl.ds(pl.multiple_of(vo[lane], EMB_K),
                                       EMB_K)], vb.at[dst], sem_v)
                return carry

            lax.fori_loop(0, CHUNK // L, body, 0)

        def drain(chunk):
            pltpu.make_async_copy(
                w_hbm.at[pl.ds(0, CHUNK * EMB_K)],
                u_bufs[chunk % 2], sem_u).wait()
            pltpu.make_async_copy(
                h_hbm.at[pl.ds(0, CHUNK * EMB_K)],
                v_bufs[chunk % 2], sem_v).wait()

        def compute(chunk):
            ub = u_bufs[chunk % 2]
            vb = v_bufs[chunk % 2]
            base = chunk * CHUNK

            def group(g, carry):
                acc = jnp.zeros((L,), jnp.float32)
                for r in range(L):
                    row_off = pl.multiple_of((g * L + r) * EMB_K, EMB_K)
                    p = (ub[pl.ds(row_off, L)] *
                         vb[pl.ds(row_off, L)])
                    for cch in range(1, K_CH):
                        sl = pl.ds(
                            pl.multiple_of(row_off + cch * L, L), L)
                        p = p + ub[sl] * vb[sl]
                    tree_buf[pl.ds(0, L)] = p
                    t = p
                    for sh in (8, 4, 2, 1):
                        t = t + tree_buf[pl.ds(sh, L)]
                        tree_buf[pl.ds(0, L)] = t
                    acc = jnp.where(
                        iota16 == r, lax.broadcast(t[0], (L,)), acc)
                out_v[pl.ds(pl.multiple_of(base + g * L, L), L)] = acc
                return carry

            lax.fori_loop(0, G_PER_CHUNK, group, 0)

        fire(0)
        fire(1)
        for chunk in range(N_CHUNKS):
            drain(chunk)
            compute(chunk)
            if chunk + 2 < N_CHUNKS:
                fire(chunk + 2)

        pltpu.sync_copy(out_v, out_hbm.at[pl.ds(base_b, B_PER_W)])

    return k(u_idx, v_idx, W_flat, H_flat)


def kernel(x, W, H, W_pre, H_pre, W_eps, H_eps):
    xi = x.astype(jnp.int32)
    u_idx = xi[:, 0]
    v_idx = xi[:, 1]
    W_flat = _to_row_major(W.T)
    H_flat = _to_row_major(H.T)
    return _mf_dot(u_idx, v_idx, W_flat, H_flat)


# speedup vs baseline: 6.2613x; 1.0586x over previous
import functools

import jax
import jax.numpy as jnp
from jax import lax
from jax.experimental import pallas as pl
from jax.experimental.pallas import tpu as pltpu
from jax.experimental.pallas import tpu_sc as plsc

BATCH = 16384
EMB_K = 64
NROWS = 100000
PANEL = 256
TBLK = 16384
N_TBLK = (NROWS + TBLK - 1) // TBLK
TROWS = N_TBLK * TBLK // 2
_INFO = plsc.get_sparse_core_info()
NC, NS, L = _INFO.num_cores, _INFO.num_subcores, _INFO.num_lanes
NW = NC * NS
B_PER_W = BATCH // NW
CHUNK = 128
N_CHUNKS = B_PER_W // CHUNK
G_PER_CHUNK = CHUNK // L
K_CH = EMB_K // L


def _to_row_major(Wt):
    def body(x_ref, o_ref):
        ident = (lax.broadcasted_iota(jnp.int32, (CHUNK, CHUNK), 0)
                 == lax.broadcasted_iota(jnp.int32, (CHUNK, CHUNK), 1)
                 ).astype(jnp.float32)
        dn = (((0,), (0,)), ((), ()))
        for p in range(TBLK // PANEL):
            lhs = jnp.concatenate(
                [x_ref[:, pl.ds(p * PANEL, CHUNK)],
                 x_ref[:, pl.ds(p * PANEL + CHUNK, CHUNK)]], axis=0)
            o_ref[pl.ds(p * CHUNK, CHUNK), :] = lax.dot_general(
                lhs, ident, dn, preferred_element_type=jnp.float32)

    out = pl.pallas_call(
        body,
        grid=(N_TBLK,),
        in_specs=[pl.BlockSpec((EMB_K, TBLK), lambda i: (0, i))],
        out_specs=pl.BlockSpec((TBLK // 2, 2 * EMB_K), lambda i: (i, 0)),
        out_shape=jax.ShapeDtypeStruct((TROWS, 2 * EMB_K), jnp.float32),
    )(Wt)
    return out.reshape(-1)


def _flat_offsets(idx):
    return (((idx >> 8) << 14) + ((idx & 127) << 7)
            + (((idx >> 7) & 1) << 6))


def _mf_dot(u_idx, v_idx, W_flat, H_flat):
    mesh = plsc.VectorSubcoreMesh(core_axis_name="c", subcore_axis_name="s")

    @functools.partial(
        pl.kernel,
        mesh=mesh,
        out_type=jax.ShapeDtypeStruct((BATCH,), jnp.float32),
        compiler_params=pltpu.CompilerParams(use_tc_tiling_on_sc=False),
        scratch_types=[
            pltpu.VMEM((B_PER_W,), jnp.int32),
            pltpu.VMEM((B_PER_W,), jnp.int32),
            pltpu.VMEM((CHUNK * EMB_K,), jnp.float32),
            pltpu.VMEM((CHUNK * EMB_K,), jnp.float32),
            pltpu.VMEM((CHUNK * EMB_K,), jnp.float32),
            pltpu.VMEM((CHUNK * EMB_K,), jnp.float32),
            pltpu.VMEM((B_PER_W,), jnp.float32),
            pltpu.VMEM((2 * L,), jnp.float32),
            pltpu.SemaphoreType.DMA,
            pltpu.SemaphoreType.DMA,
        ],
    )
    def k(u_idx_hbm, v_idx_hbm, w_hbm, h_hbm, out_hbm,
          idx_u_v, idx_v_v, u_buf0, u_buf1, v_buf0, v_buf1,
          out_v, tree_buf, sem_u, sem_v):
        c = lax.axis_index("c")
        s = lax.axis_index("s")
        wid = s * NC + c
        base_b = pl.multiple_of(wid * B_PER_W, B_PER_W)

        pltpu.sync_copy(u_idx_hbm.at[pl.ds(base_b, B_PER_W)], idx_u_v)
        pltpu.sync_copy(v_idx_hbm.at[pl.ds(base_b, B_PER_W)], idx_v_v)

        iota16 = lax.iota(jnp.int32, L)
        tree_buf[pl.ds(L, L)] = jnp.zeros((L,), jnp.float32)

        u_bufs = (u_buf0, u_buf1)
        v_bufs = (v_buf0, v_buf1)

        def fire(chunk):
            ub = u_bufs[chunk % 2]
            vb = v_bufs[chunk % 2]
            base = chunk * CHUNK

            def body(g16, carry):
                off = pl.multiple_of(g16 * L, L)
                uo = _flat_offsets(idx_u_v[pl.ds(base + off, L)])
                vo = _flat_offsets(idx_v_v[pl.ds(base + off, L)])
                for lane in range(L):
                    dst = pl.ds(
                        pl.multiple_of((off + lane) * EMB_K, EMB_K), EMB_K)
                    pltpu.async_copy(
                        w_hbm.at[pl.ds(pl.multiple_of(uo[lane], EMB_K),
                                       EMB_K)], ub.at[dst], sem_u)
                    pltpu.async_copy(
                        h_hbm.at[pl.ds(pl.multiple_of(vo[lane], EMB_K),
                                       EMB_K)], vb.at[dst], sem_v)
                return carry

            lax.fori_loop(0, CHUNK // L, body, 0)

        def drain(chunk):
            pltpu.make_async_copy(
                w_hbm.at[pl.ds(0, CHUNK * EMB_K)],
                u_bufs[chunk % 2], sem_u).wait()
            pltpu.make_async_copy(
                h_hbm.at[pl.ds(0, CHUNK * EMB_K)],
                v_bufs[chunk % 2], sem_v).wait()

        def compute(chunk):
            ub = u_bufs[chunk % 2]
            vb = v_bufs[chunk % 2]
            base = chunk * CHUNK

            def group(g, carry):
                acc = jnp.zeros((L,), jnp.float32)
                for r in range(L):
                    row_off = pl.multiple_of((g * L + r) * EMB_K, EMB_K)
                    p = (ub[pl.ds(row_off, L)] *
                         vb[pl.ds(row_off, L)])
                    for cch in range(1, K_CH):
                        sl = pl.ds(
                            pl.multiple_of(row_off + cch * L, L), L)
                        p = p + ub[sl] * vb[sl]
                    tree_buf[pl.ds(0, L)] = p
                    t = p
                    for sh in (8, 4, 2, 1):
                        t = t + tree_buf[pl.ds(sh, L)]
                        tree_buf[pl.ds(0, L)] = t
                    acc = jnp.where(
                        iota16 == r, lax.broadcast(t[0], (L,)), acc)
                out_v[pl.ds(pl.multiple_of(base + g * L, L), L)] = acc
                return carry

            lax.fori_loop(0, G_PER_CHUNK, group, 0)

        fire(0)
        fire(1)
        for chunk in range(N_CHUNKS):
            drain(chunk)
            compute(chunk)
            if chunk + 2 < N_CHUNKS:
                fire(chunk + 2)

        pltpu.sync_copy(out_v, out_hbm.at[pl.ds(base_b, B_PER_W)])

    return k(u_idx, v_idx, W_flat, H_flat)


def kernel(x, W, H, W_pre, H_pre, W_eps, H_eps):
    xi = x.astype(jnp.int32)
    u_idx = xi[:, 0]
    v_idx = xi[:, 1]
    W_flat = _to_row_major(W.T)
    H_flat = _to_row_major(H.T)
    return _mf_dot(u_idx, v_idx, W_flat, H_flat)


# speedup vs baseline: 6.3545x; 1.0149x over previous
import functools

import jax
import jax.numpy as jnp
from jax import lax
from jax.experimental import pallas as pl
from jax.experimental.pallas import tpu as pltpu
from jax.experimental.pallas import tpu_sc as plsc

BATCH = 16384
EMB_K = 64
NROWS = 100000
PANEL = 256
TBLK = 25088
N_TBLK = (NROWS + TBLK - 1) // TBLK
TROWS = N_TBLK * TBLK // 2
_INFO = plsc.get_sparse_core_info()
NC, NS, L = _INFO.num_cores, _INFO.num_subcores, _INFO.num_lanes
NW = NC * NS
B_PER_W = BATCH // NW
CHUNK = 128
N_CHUNKS = B_PER_W // CHUNK
G_PER_CHUNK = CHUNK // L
K_CH = EMB_K // L


def _to_row_major(Wt):
    def body(x_ref, o_ref):
        ident = (lax.broadcasted_iota(jnp.int32, (CHUNK, CHUNK), 0)
                 == lax.broadcasted_iota(jnp.int32, (CHUNK, CHUNK), 1)
                 ).astype(jnp.float32)
        dn = (((0,), (0,)), ((), ()))
        for p in range(TBLK // PANEL):
            lhs = jnp.concatenate(
                [x_ref[:, pl.ds(p * PANEL, CHUNK)],
                 x_ref[:, pl.ds(p * PANEL + CHUNK, CHUNK)]], axis=0)
            o_ref[pl.ds(p * CHUNK, CHUNK), :] = lax.dot_general(
                lhs, ident, dn, preferred_element_type=jnp.float32)

    out = pl.pallas_call(
        body,
        grid=(N_TBLK,),
        in_specs=[pl.BlockSpec((EMB_K, TBLK), lambda i: (0, i))],
        out_specs=pl.BlockSpec((TBLK // 2, 2 * EMB_K), lambda i: (i, 0)),
        out_shape=jax.ShapeDtypeStruct((TROWS, 2 * EMB_K), jnp.float32),
    )(Wt)
    return out.reshape(-1)


def _flat_offsets(idx):
    return (((idx >> 8) << 14) + ((idx & 127) << 7)
            + (((idx >> 7) & 1) << 6))


def _mf_dot(u_idx, v_idx, W_flat, H_flat):
    mesh = plsc.VectorSubcoreMesh(core_axis_name="c", subcore_axis_name="s")

    @functools.partial(
        pl.kernel,
        mesh=mesh,
        out_type=jax.ShapeDtypeStruct((BATCH,), jnp.float32),
        compiler_params=pltpu.CompilerParams(use_tc_tiling_on_sc=False),
        scratch_types=[
            pltpu.VMEM((B_PER_W,), jnp.int32),
            pltpu.VMEM((B_PER_W,), jnp.int32),
            pltpu.VMEM((CHUNK * EMB_K,), jnp.float32),
            pltpu.VMEM((CHUNK * EMB_K,), jnp.float32),
            pltpu.VMEM((CHUNK * EMB_K,), jnp.float32),
            pltpu.VMEM((CHUNK * EMB_K,), jnp.float32),
            pltpu.VMEM((B_PER_W,), jnp.float32),
            pltpu.VMEM((2 * L,), jnp.float32),
            pltpu.SemaphoreType.DMA,
            pltpu.SemaphoreType.DMA,
        ],
    )
    def k(u_idx_hbm, v_idx_hbm, w_hbm, h_hbm, out_hbm,
          idx_u_v, idx_v_v, u_buf0, u_buf1, v_buf0, v_buf1,
          out_v, tree_buf, sem_u, sem_v):
        c = lax.axis_index("c")
        s = lax.axis_index("s")
        wid = s * NC + c
        base_b = pl.multiple_of(wid * B_PER_W, B_PER_W)

        pltpu.sync_copy(u_idx_hbm.at[pl.ds(base_b, B_PER_W)], idx_u_v)
        pltpu.sync_copy(v_idx_hbm.at[pl.ds(base_b, B_PER_W)], idx_v_v)

        iota16 = lax.iota(jnp.int32, L)
        tree_buf[pl.ds(L, L)] = jnp.zeros((L,), jnp.float32)

        u_bufs = (u_buf0, u_buf1)
        v_bufs = (v_buf0, v_buf1)

        def fire(chunk):
            ub = u_bufs[chunk % 2]
            vb = v_bufs[chunk % 2]
            base = chunk * CHUNK

            def body(g16, carry):
                off = pl.multiple_of(g16 * L, L)
                uo = _flat_offsets(idx_u_v[pl.ds(base + off, L)])
                vo = _flat_offsets(idx_v_v[pl.ds(base + off, L)])
                for lane in range(L):
                    dst = pl.ds(
                        pl.multiple_of((off + lane) * EMB_K, EMB_K), EMB_K)
                    pltpu.async_copy(
                        w_hbm.at[pl.ds(pl.multiple_of(uo[lane], EMB_K),
                                       EMB_K)], ub.at[dst], sem_u)
                    pltpu.async_copy(
                        h_hbm.at[pl.ds(pl.multiple_of(vo[lane], EMB_K),
                                       EMB_K)], vb.at[dst], sem_v)
                return carry

            lax.fori_loop(0, CHUNK // L, body, 0)

        def drain(chunk):
            pltpu.make_async_copy(
                w_hbm.at[pl.ds(0, CHUNK * EMB_K)],
                u_bufs[chunk % 2], sem_u).wait()
            pltpu.make_async_copy(
                h_hbm.at[pl.ds(0, CHUNK * EMB_K)],
                v_bufs[chunk % 2], sem_v).wait()

        def compute(chunk):
            ub = u_bufs[chunk % 2]
            vb = v_bufs[chunk % 2]
            base = chunk * CHUNK

            def group(g, carry):
                acc = jnp.zeros((L,), jnp.float32)
                for r in range(L):
                    row_off = pl.multiple_of((g * L + r) * EMB_K, EMB_K)
                    p = (ub[pl.ds(row_off, L)] *
                         vb[pl.ds(row_off, L)])
                    for cch in range(1, K_CH):
                        sl = pl.ds(
                            pl.multiple_of(row_off + cch * L, L), L)
                        p = p + ub[sl] * vb[sl]
                    tree_buf[pl.ds(0, L)] = p
                    t = p
                    for sh in (8, 4, 2, 1):
                        t = t + tree_buf[pl.ds(sh, L)]
                        tree_buf[pl.ds(0, L)] = t
                    acc = jnp.where(
                        iota16 == r, lax.broadcast(t[0], (L,)), acc)
                out_v[pl.ds(pl.multiple_of(base + g * L, L), L)] = acc
                return carry

            lax.fori_loop(0, G_PER_CHUNK, group, 0)

        fire(0)
        fire(1)
        for chunk in range(N_CHUNKS):
            drain(chunk)
            compute(chunk)
            if chunk + 2 < N_CHUNKS:
                fire(chunk + 2)

        pltpu.sync_copy(out_v, out_hbm.at[pl.ds(base_b, B_PER_W)])

    return k(u_idx, v_idx, W_flat, H_flat)


def kernel(x, W, H, W_pre, H_pre, W_eps, H_eps):
    xi = x.astype(jnp.int32)
    u_idx = xi[:, 0]
    v_idx = xi[:, 1]
    W_flat = _to_row_major(W.T)
    H_flat = _to_row_major(H.T)
    return _mf_dot(u_idx, v_idx, W_flat, H_flat)


# speedup vs baseline: 6.7496x; 1.0622x over previous
import functools

import jax
import jax.numpy as jnp
from jax import lax
from jax.experimental import pallas as pl
from jax.experimental.pallas import tpu as pltpu
from jax.experimental.pallas import tpu_sc as plsc

BATCH = 16384
EMB_K = 64
NROWS = 100000
PANEL = 256
TBLK = 50176
N_TBLK = (NROWS + TBLK - 1) // TBLK
TROWS = N_TBLK * TBLK // 2
_INFO = plsc.get_sparse_core_info()
NC, NS, L = _INFO.num_cores, _INFO.num_subcores, _INFO.num_lanes
NW = NC * NS
B_PER_W = BATCH // NW
CHUNK = 128
N_CHUNKS = B_PER_W // CHUNK
G_PER_CHUNK = CHUNK // L
K_CH = EMB_K // L


def _to_row_major(Wt):
    def body(x_ref, o_ref):
        ident = (lax.broadcasted_iota(jnp.int32, (CHUNK, CHUNK), 0)
                 == lax.broadcasted_iota(jnp.int32, (CHUNK, CHUNK), 1)
                 ).astype(jnp.float32)
        dn = (((0,), (0,)), ((), ()))
        for p in range(TBLK // PANEL):
            lhs = jnp.concatenate(
                [x_ref[:, pl.ds(p * PANEL, CHUNK)],
                 x_ref[:, pl.ds(p * PANEL + CHUNK, CHUNK)]], axis=0)
            o_ref[pl.ds(p * CHUNK, CHUNK), :] = lax.dot_general(
                lhs, ident, dn, preferred_element_type=jnp.float32)

    out = pl.pallas_call(
        body,
        grid=(N_TBLK,),
        in_specs=[pl.BlockSpec((EMB_K, TBLK), lambda i: (0, i))],
        out_specs=pl.BlockSpec((TBLK // 2, 2 * EMB_K), lambda i: (i, 0)),
        out_shape=jax.ShapeDtypeStruct((TROWS, 2 * EMB_K), jnp.float32),
    )(Wt)
    return out.reshape(-1)


def _flat_offsets(idx):
    return (((idx >> 8) << 14) + ((idx & 127) << 7)
            + (((idx >> 7) & 1) << 6))


def _mf_dot(u_idx, v_idx, W_flat, H_flat):
    mesh = plsc.VectorSubcoreMesh(core_axis_name="c", subcore_axis_name="s")

    @functools.partial(
        pl.kernel,
        mesh=mesh,
        out_type=jax.ShapeDtypeStruct((BATCH,), jnp.float32),
        compiler_params=pltpu.CompilerParams(use_tc_tiling_on_sc=False),
        scratch_types=[
            pltpu.VMEM((B_PER_W,), jnp.int32),
            pltpu.VMEM((B_PER_W,), jnp.int32),
            pltpu.VMEM((CHUNK * EMB_K,), jnp.float32),
            pltpu.VMEM((CHUNK * EMB_K,), jnp.float32),
            pltpu.VMEM((CHUNK * EMB_K,), jnp.float32),
            pltpu.VMEM((CHUNK * EMB_K,), jnp.float32),
            pltpu.VMEM((B_PER_W,), jnp.float32),
            pltpu.VMEM((2 * L,), jnp.float32),
            pltpu.SemaphoreType.DMA,
            pltpu.SemaphoreType.DMA,
        ],
    )
    def k(u_idx_hbm, v_idx_hbm, w_hbm, h_hbm, out_hbm,
          idx_u_v, idx_v_v, u_buf0, u_buf1, v_buf0, v_buf1,
          out_v, tree_buf, sem_u, sem_v):
        c = lax.axis_index("c")
        s = lax.axis_index("s")
        wid = s * NC + c
        base_b = pl.multiple_of(wid * B_PER_W, B_PER_W)

        pltpu.sync_copy(u_idx_hbm.at[pl.ds(base_b, B_PER_W)], idx_u_v)
        pltpu.sync_copy(v_idx_hbm.at[pl.ds(base_b, B_PER_W)], idx_v_v)

        iota16 = lax.iota(jnp.int32, L)
        tree_buf[pl.ds(L, L)] = jnp.zeros((L,), jnp.float32)

        u_bufs = (u_buf0, u_buf1)
        v_bufs = (v_buf0, v_buf1)

        def fire(chunk):
            ub = u_bufs[chunk % 2]
            vb = v_bufs[chunk % 2]
            base = chunk * CHUNK

            def body(g16, carry):
                off = pl.multiple_of(g16 * L, L)
                uo = _flat_offsets(idx_u_v[pl.ds(base + off, L)])
                vo = _flat_offsets(idx_v_v[pl.ds(base + off, L)])
                for lane in range(L):
                    dst = pl.ds(
                        pl.multiple_of((off + lane) * EMB_K, EMB_K), EMB_K)
                    pltpu.async_copy(
                        w_hbm.at[pl.ds(pl.multiple_of(uo[lane], EMB_K),
                                       EMB_K)], ub.at[dst], sem_u)
                    pltpu.async_copy(
                        h_hbm.at[pl.ds(pl.multiple_of(vo[lane], EMB_K),
                                       EMB_K)], vb.at[dst], sem_v)
                return carry

            lax.fori_loop(0, CHUNK // L, body, 0)

        def drain(chunk):
            pltpu.make_async_copy(
                w_hbm.at[pl.ds(0, CHUNK * EMB_K)],
                u_bufs[chunk % 2], sem_u).wait()
            pltpu.make_async_copy(
                h_hbm.at[pl.ds(0, CHUNK * EMB_K)],
                v_bufs[chunk % 2], sem_v).wait()

        def compute(chunk):
            ub = u_bufs[chunk % 2]
            vb = v_bufs[chunk % 2]
            base = chunk * CHUNK

            def group(g, carry):
                acc = jnp.zeros((L,), jnp.float32)
                for r in range(L):
                    row_off = pl.multiple_of((g * L + r) * EMB_K, EMB_K)
                    p = (ub[pl.ds(row_off, L)] *
                         vb[pl.ds(row_off, L)])
                    for cch in range(1, K_CH):
                        sl = pl.ds(
                            pl.multiple_of(row_off + cch * L, L), L)
                        p = p + ub[sl] * vb[sl]
                    tree_buf[pl.ds(0, L)] = p
                    t = p
                    for sh in (8, 4, 2, 1):
                        t = t + tree_buf[pl.ds(sh, L)]
                        tree_buf[pl.ds(0, L)] = t
                    acc = jnp.where(
                        iota16 == r, lax.broadcast(t[0], (L,)), acc)
                out_v[pl.ds(pl.multiple_of(base + g * L, L), L)] = acc
                return carry

            lax.fori_loop(0, G_PER_CHUNK, group, 0)

        fire(0)
        fire(1)
        for chunk in range(N_CHUNKS):
            drain(chunk)
            compute(chunk)
            if chunk + 2 < N_CHUNKS:
                fire(chunk + 2)

        pltpu.sync_copy(out_v, out_hbm.at[pl.ds(base_b, B_PER_W)])

    return k(u_idx, v_idx, W_flat, H_flat)


def kernel(x, W, H, W_pre, H_pre, W_eps, H_eps):
    xi = x.astype(jnp.int32)
    u_idx = xi[:, 0]
    v_idx = xi[:, 1]
    W_flat = _to_row_major(W.T)
    H_flat = _to_row_major(H.T)
    return _mf_dot(u_idx, v_idx, W_flat, H_flat)


# speedup vs baseline: 7.6168x; 1.1285x over previous
import functools

import jax
import jax.numpy as jnp
from jax import lax
from jax.experimental import pallas as pl
from jax.experimental.pallas import tpu as pltpu
from jax.experimental.pallas import tpu_sc as plsc

BATCH = 16384
EMB_K = 64
NROWS = 100000
PANEL = 256
TBLK = 50176
N_TBLK = (NROWS + TBLK - 1) // TBLK
TROWS = N_TBLK * TBLK // 2
_INFO = plsc.get_sparse_core_info()
NC, NS, L = _INFO.num_cores, _INFO.num_subcores, _INFO.num_lanes
NW = NC * NS
B_PER_W = BATCH // NW
CHUNK = 128
N_CHUNKS = B_PER_W // CHUNK
G_PER_CHUNK = CHUNK // L
K_CH = EMB_K // L


def _to_row_major(Wt):
    def body(x_ref, o_ref):
        ident = (lax.broadcasted_iota(jnp.int32, (CHUNK, CHUNK), 0)
                 == lax.broadcasted_iota(jnp.int32, (CHUNK, CHUNK), 1)
                 ).astype(jnp.float32)
        dn = (((0,), (0,)), ((), ()))
        for p in range(TBLK // PANEL):
            lhs = jnp.concatenate(
                [x_ref[:, pl.ds(p * PANEL, CHUNK)],
                 x_ref[:, pl.ds(p * PANEL + CHUNK, CHUNK)]], axis=0)
            o_ref[pl.ds(p * CHUNK, CHUNK), :] = lax.dot_general(
                lhs, ident, dn, preferred_element_type=jnp.float32)

    out = pl.pallas_call(
        body,
        grid=(N_TBLK,),
        in_specs=[pl.BlockSpec((EMB_K, TBLK), lambda i: (0, i))],
        out_specs=pl.BlockSpec((TBLK // 2, 2 * EMB_K), lambda i: (i, 0)),
        out_shape=jax.ShapeDtypeStruct((TROWS, 2 * EMB_K), jnp.float32),
    )(Wt)
    return out.reshape(-1)


def _flat_offsets(idx):
    return (((idx >> 8) << 14) + ((idx & 127) << 7)
            + (((idx >> 7) & 1) << 6))


def _mf_dot(u_idx, v_idx, W_flat, H_flat):
    mesh = plsc.VectorSubcoreMesh(core_axis_name="c", subcore_axis_name="s")

    @functools.partial(
        pl.kernel,
        mesh=mesh,
        out_type=jax.ShapeDtypeStruct((BATCH,), jnp.float32),
        compiler_params=pltpu.CompilerParams(use_tc_tiling_on_sc=False),
        scratch_types=[
            pltpu.VMEM((B_PER_W,), jnp.int32),
            pltpu.VMEM((B_PER_W,), jnp.int32),
            pltpu.VMEM((CHUNK * EMB_K,), jnp.float32),
            pltpu.VMEM((CHUNK * EMB_K,), jnp.float32),
            pltpu.VMEM((CHUNK * EMB_K,), jnp.float32),
            pltpu.VMEM((CHUNK * EMB_K,), jnp.float32),
            pltpu.VMEM((B_PER_W,), jnp.float32),
            pltpu.VMEM((2 * L * L,), jnp.float32),
            pltpu.SemaphoreType.DMA,
            pltpu.SemaphoreType.DMA,
        ],
    )
    def k(u_idx_hbm, v_idx_hbm, w_hbm, h_hbm, out_hbm,
          idx_u_v, idx_v_v, u_buf0, u_buf1, v_buf0, v_buf1,
          out_v, tree_buf, sem_u, sem_v):
        c = lax.axis_index("c")
        s = lax.axis_index("s")
        wid = s * NC + c
        base_b = pl.multiple_of(wid * B_PER_W, B_PER_W)

        pltpu.sync_copy(u_idx_hbm.at[pl.ds(base_b, B_PER_W)], idx_u_v)
        pltpu.sync_copy(v_idx_hbm.at[pl.ds(base_b, B_PER_W)], idx_v_v)

        iota16 = lax.iota(jnp.int32, L)
        for r in range(L):
            tree_buf[pl.ds(r * 2 * L + L, L)] = jnp.zeros((L,), jnp.float32)

        u_bufs = (u_buf0, u_buf1)
        v_bufs = (v_buf0, v_buf1)

        def fire(chunk):
            ub = u_bufs[chunk % 2]
            vb = v_bufs[chunk % 2]
            base = chunk * CHUNK

            def body(g16, carry):
                off = pl.multiple_of(g16 * L, L)
                uo = _flat_offsets(idx_u_v[pl.ds(base + off, L)])
                vo = _flat_offsets(idx_v_v[pl.ds(base + off, L)])
                for lane in range(L):
                    dst = pl.ds(
                        pl.multiple_of((off + lane) * EMB_K, EMB_K), EMB_K)
                    pltpu.async_copy(
                        w_hbm.at[pl.ds(pl.multiple_of(uo[lane], EMB_K),
                                       EMB_K)], ub.at[dst], sem_u)
                    pltpu.async_copy(
                        h_hbm.at[pl.ds(pl.multiple_of(vo[lane], EMB_K),
                                       EMB_K)], vb.at[dst], sem_v)
                return carry

            lax.fori_loop(0, CHUNK // L, body, 0)

        def drain(chunk):
            pltpu.make_async_copy(
                w_hbm.at[pl.ds(0, CHUNK * EMB_K)],
                u_bufs[chunk % 2], sem_u).wait()
            pltpu.make_async_copy(
                h_hbm.at[pl.ds(0, CHUNK * EMB_K)],
                v_bufs[chunk % 2], sem_v).wait()

        def compute(chunk):
            ub = u_bufs[chunk % 2]
            vb = v_bufs[chunk % 2]
            base = chunk * CHUNK

            def group(g, carry):
                ts = []
                for r in range(L):
                    row_off = pl.multiple_of((g * L + r) * EMB_K, EMB_K)
                    p = (ub[pl.ds(row_off, L)] *
                         vb[pl.ds(row_off, L)])
                    for cch in range(1, K_CH):
                        sl = pl.ds(
                            pl.multiple_of(row_off + cch * L, L), L)
                        p = p + ub[sl] * vb[sl]
                    tree_buf[pl.ds(r * 2 * L, L)] = p
                    ts.append(p)
                for sh in (8, 4, 2, 1):
                    for r in range(L):
                        ts[r] = ts[r] + tree_buf[pl.ds(r * 2 * L + sh, L)]
                        if sh != 1:
                            tree_buf[pl.ds(r * 2 * L, L)] = ts[r]
                acc = jnp.zeros((L,), jnp.float32)
                for r in range(L):
                    acc = jnp.where(
                        iota16 == r, lax.broadcast(ts[r][0], (L,)), acc)
                out_v[pl.ds(pl.multiple_of(base + g * L, L), L)] = acc
                return carry

            lax.fori_loop(0, G_PER_CHUNK, group, 0)

        fire(0)
        fire(1)
        for chunk in range(N_CHUNKS):
            drain(chunk)
            compute(chunk)
            if chunk + 2 < N_CHUNKS:
                fire(chunk + 2)

        pltpu.sync_copy(out_v, out_hbm.at[pl.ds(base_b, B_PER_W)])

    return k(u_idx, v_idx, W_flat, H_flat)


def kernel(x, W, H, W_pre, H_pre, W_eps, H_eps):
    xi = x.astype(jnp.int32)
    u_idx = xi[:, 0]
    v_idx = xi[:, 1]
    W_flat = _to_row_major(W.T)
    H_flat = _to_row_major(H.T)
    return _mf_dot(u_idx, v_idx, W_flat, H_flat)
